# baseline (speedup 1.0000x reference)
import functools

import jax
import jax.numpy as jnp
from jax import lax
from jax.experimental import pallas as pl
from jax.experimental.pallas import tpu as pltpu
from jax.experimental.pallas import tpu_sc as plsc

NC = 2
NS = 16
NW = NC * NS
L = 16

R = 128
F_IN = 126
F_OUT = 258
N_POKE = 6
RAW = 15

_TABLE_SPECS = (
    ("name", 8, 15, 0),
    ("status", 4, 23, 6),
    ("type1", 4, 27, 12),
    ("type2", 4, 31, 18),
    ("ability", 4, 35, 24),
    ("item", 4, 39, 30),
)


def _make_sc_call(n_rows):
    assert n_rows % (NW * R) == 0
    rows_per_tile = n_rows // NW
    n_chunks = rows_per_tile // R
    mesh = plsc.VectorSubcoreMesh(
        core_axis_name="c", subcore_axis_name="s",
        num_cores=NC, num_subcores=NS)

    @functools.partial(
        pl.kernel,
        mesh=mesh,
        compiler_params=pltpu.CompilerParams(
            use_tc_tiling_on_sc=False, needs_layout_passes=False),
        out_type=jax.ShapeDtypeStruct((n_rows * F_OUT,), jnp.float32),
        scratch_types=[
            pltpu.VMEM((R * F_IN,), jnp.float32),
            pltpu.VMEM((R * F_OUT,), jnp.float32),
            pltpu.VMEM((2048 * 8,), jnp.float32),
            pltpu.VMEM((8 * 4,), jnp.float32),
            pltpu.VMEM((32 * 4,), jnp.float32),
            pltpu.VMEM((512 * 4,), jnp.float32),
            pltpu.VMEM((1024 * 4,), jnp.float32),
        ],
    )
    def sc_call(x_hbm, name_hbm, stat_hbm, type_hbm, abil_hbm, item_hbm,
                out_hbm, x_v, out_v, name_v, stat_v, type_v, abil_v, item_v):
        wid = lax.axis_index("s") * NC + lax.axis_index("c")
        base_row = wid * rows_per_tile

        pltpu.sync_copy(name_hbm, name_v)
        pltpu.sync_copy(stat_hbm, stat_v)
        pltpu.sync_copy(type_hbm, type_v)
        pltpu.sync_copy(abil_hbm, abil_v)
        pltpu.sync_copy(item_hbm, item_v)
        tables = {"name": name_v, "status": stat_v, "type1": type_v,
                  "type2": type_v, "ability": abil_v, "item": item_v}

        def group_body(g):
            rows = g * L + lax.iota(jnp.int32, L)
            rb_in = rows * F_IN
            rb_out = rows * F_OUT
            for p in range(N_POKE):
                obase = p * (RAW + 28)
                for j in range(RAW):
                    v = plsc.load_gather(x_v, [rb_in + (36 + RAW * p + j)])
                    plsc.store_scatter(out_v, [rb_out + (obase + j)], v)
                for tname, dim, off, colbase in _TABLE_SPECS:
                    fidx = plsc.load_gather(x_v, [rb_in + (colbase + p)])
                    tbase = fidx.astype(jnp.int32) * dim
                    for d in range(dim):
                        v = plsc.load_gather(tables[tname], [tbase + d])
                        plsc.store_scatter(
                            out_v, [rb_out + (obase + off + d)], v)

        def chunk_body(ci, _):
            row0 = base_row + ci * R
            in0 = pl.multiple_of(row0 * F_IN, 8)
            out0 = pl.multiple_of(row0 * F_OUT, 8)
            pltpu.sync_copy(x_hbm.at[pl.ds(in0, R * F_IN)], x_v)
            plsc.parallel_loop(0, R // L, 1, unroll=2)(group_body)
            pltpu.sync_copy(out_v, out_hbm.at[pl.ds(out0, R * F_OUT)])
            return 0

        lax.fori_loop(0, n_chunks, chunk_body, 0)

    return sc_call


def kernel(x, name_table, status_table, type_table, ability_table, item_table):
    B, T, F = x.shape
    n_rows = B * T
    sc_call = _make_sc_call(n_rows)
    out = sc_call(x.reshape(-1), name_table.reshape(-1),
                  status_table.reshape(-1), type_table.reshape(-1),
                  ability_table.reshape(-1), item_table.reshape(-1))
    return out.reshape(B, T, F_OUT)


# speedup vs baseline: 1.0902x; 1.0902x over previous
import functools

import jax
import jax.numpy as jnp
from jax import lax
from jax.experimental import pallas as pl
from jax.experimental.pallas import tpu as pltpu
from jax.experimental.pallas import tpu_sc as plsc

NC = 2
NS = 16
NW = NC * NS
L = 16

R = 128
F_IN = 126
F_OUT = 258
N_POKE = 6
RAW = 15

_TABLE_SPECS = (
    ("name", 8, 15, 0),
    ("status", 4, 23, 6),
    ("type1", 4, 27, 12),
    ("type2", 4, 31, 18),
    ("ability", 4, 35, 24),
    ("item", 4, 39, 30),
)


def _make_sc_call(n_rows):
    assert n_rows % (NW * R) == 0
    rows_per_tile = n_rows // NW
    n_chunks = rows_per_tile // R
    mesh = plsc.VectorSubcoreMesh(
        core_axis_name="c", subcore_axis_name="s",
        num_cores=NC, num_subcores=NS)

    @functools.partial(
        pl.kernel,
        mesh=mesh,
        compiler_params=pltpu.CompilerParams(
            use_tc_tiling_on_sc=False, needs_layout_passes=False),
        out_type=jax.ShapeDtypeStruct((n_rows * F_OUT,), jnp.float32),
        scratch_types=[
            pltpu.VMEM((R * F_IN,), jnp.float32),
            pltpu.VMEM((R * F_OUT,), jnp.float32),
            pltpu.VMEM((2048 * 8,), jnp.float32),
            pltpu.VMEM((8 * 4,), jnp.float32),
            pltpu.VMEM((32 * 4,), jnp.float32),
            pltpu.VMEM((512 * 4,), jnp.float32),
            pltpu.VMEM((1024 * 4,), jnp.float32),
        ],
    )
    def sc_call(x_hbm, name_hbm, stat_hbm, type_hbm, abil_hbm, item_hbm,
                out_hbm, x_v, out_v, name_v, stat_v, type_v, abil_v, item_v):
        wid = lax.axis_index("s") * NC + lax.axis_index("c")
        base_row = wid * rows_per_tile

        pltpu.sync_copy(name_hbm, name_v)
        pltpu.sync_copy(stat_hbm, stat_v)
        pltpu.sync_copy(type_hbm, type_v)
        pltpu.sync_copy(abil_hbm, abil_v)
        pltpu.sync_copy(item_hbm, item_v)
        tables = {"name": name_v, "status": stat_v, "type1": type_v,
                  "type2": type_v, "ability": abil_v, "item": item_v}

        def group_body(g):
            rows = g * L + lax.iota(jnp.int32, L)
            rb_in = rows * F_IN
            rb_out = rows * F_OUT
            for p in range(N_POKE):
                obase = p * (RAW + 28)
                for j in range(RAW):
                    v = plsc.load_gather(x_v, [rb_in + (36 + RAW * p + j)])
                    plsc.store_scatter(out_v, [rb_out + (obase + j)], v)
                for tname, dim, off, colbase in _TABLE_SPECS:
                    fidx = plsc.load_gather(x_v, [rb_in + (colbase + p)])
                    tbase = fidx.astype(jnp.int32) * dim
                    for d in range(dim):
                        v = plsc.load_gather(tables[tname], [tbase + d])
                        plsc.store_scatter(
                            out_v, [rb_out + (obase + off + d)], v)

        def chunk_body(ci, _):
            row0 = base_row + ci * R
            in0 = pl.multiple_of(row0 * F_IN, 8)
            out0 = pl.multiple_of(row0 * F_OUT, 8)
            pltpu.sync_copy(x_hbm.at[pl.ds(in0, R * F_IN)], x_v)
            plsc.parallel_loop(0, R // L, 1)(group_body)
            pltpu.sync_copy(out_v, out_hbm.at[pl.ds(out0, R * F_OUT)])
            return 0

        lax.fori_loop(0, n_chunks, chunk_body, 0)

    return sc_call


def kernel(x, name_table, status_table, type_table, ability_table, item_table):
    B, T, F = x.shape
    n_rows = B * T
    sc_call = _make_sc_call(n_rows)
    out = sc_call(x.reshape(-1), name_table.reshape(-1),
                  status_table.reshape(-1), type_table.reshape(-1),
                  ability_table.reshape(-1), item_table.reshape(-1))
    return out.reshape(B, T, F_OUT)


# speedup vs baseline: 1.4517x; 1.3316x over previous
import functools

import jax
import jax.numpy as jnp
from jax import lax
from jax.experimental import pallas as pl
from jax.experimental.pallas import tpu as pltpu
from jax.experimental.pallas import tpu_sc as plsc

NC = 2
NS = 16
NW = NC * NS
L = 16

R = 128
F_IN = 126
F_OUT = 258
N_POKE = 6
RAW = 15
POKE_W = 43
WINDOW = 10

_TABLE_SPECS = (
    ("name", 8, 15, 0),
    ("status", 4, 23, 6),
    ("type1", 4, 27, 12),
    ("type2", 4, 31, 18),
    ("ability", 4, 35, 24),
    ("item", 4, 39, 30),
)


def _make_sc_call(n_rows):
    assert n_rows % (NW * R) == 0
    rows_per_tile = n_rows // NW
    n_chunks = rows_per_tile // R
    mesh = plsc.VectorSubcoreMesh(
        core_axis_name="c", subcore_axis_name="s",
        num_cores=NC, num_subcores=NS)

    @functools.partial(
        pl.kernel,
        mesh=mesh,
        compiler_params=pltpu.CompilerParams(
            use_tc_tiling_on_sc=False, needs_layout_passes=False),
        out_type=jax.ShapeDtypeStruct((n_rows * F_OUT,), jnp.float32),
        scratch_types=[
            pltpu.VMEM((R * F_IN,), jnp.float32),
            pltpu.VMEM((R * F_OUT,), jnp.float32),
            pltpu.VMEM((2048 * 8,), jnp.float32),
            pltpu.VMEM((8 * 4,), jnp.float32),
            pltpu.VMEM((32 * 4,), jnp.float32),
            pltpu.VMEM((512 * 4,), jnp.float32),
            pltpu.VMEM((1024 * 4,), jnp.float32),
        ],
    )
    def sc_call(x_hbm, name_hbm, stat_hbm, type_hbm, abil_hbm, item_hbm,
                out_hbm, x_v, out_v, name_v, stat_v, type_v, abil_v, item_v):
        wid = lax.axis_index("s") * NC + lax.axis_index("c")
        base_row = wid * rows_per_tile

        pltpu.sync_copy(name_hbm, name_v)
        pltpu.sync_copy(stat_hbm, stat_v)
        pltpu.sync_copy(type_hbm, type_v)
        pltpu.sync_copy(abil_hbm, abil_v)
        pltpu.sync_copy(item_hbm, item_v)
        tables = {"name": name_v, "status": stat_v, "type1": type_v,
                  "type2": type_v, "ability": abil_v, "item": item_v}

        def group_body(g):
            rows = g * L + lax.iota(jnp.int32, L)
            rb_in = rows * F_IN
            rb_out = rows * F_OUT
            for p in range(N_POKE):
                obase = p * POKE_W
                fidx = [plsc.load_gather(x_v, [rb_in + (cb + p)])
                        for (_, _, _, cb) in _TABLE_SPECS]
                tbase = [fidx[k].astype(jnp.int32) * dim
                         for k, (_, dim, _, _) in enumerate(_TABLE_SPECS)]
                work = [(x_v, rb_in + (F_IN - RAW * N_POKE + RAW * p + j),
                         obase + j) for j in range(RAW)]
                for k, (tname, dim, off, _) in enumerate(_TABLE_SPECS):
                    work += [(tables[tname], tbase[k] + d, obase + off + d)
                             for d in range(dim)]
                for w0 in range(0, len(work), WINDOW):
                    batch = work[w0:w0 + WINDOW]
                    vals = [plsc.load_gather(ref, [src])
                            for ref, src, _ in batch]
                    for (_, _, dst), v in zip(batch, vals):
                        plsc.store_scatter(out_v, [rb_out + dst], v)

        def chunk_body(ci, _):
            row0 = base_row + ci * R
            in0 = pl.multiple_of(row0 * F_IN, 8)
            out0 = pl.multiple_of(row0 * F_OUT, 8)
            pltpu.sync_copy(x_hbm.at[pl.ds(in0, R * F_IN)], x_v)
            plsc.parallel_loop(0, R // L, 1)(group_body)
            pltpu.sync_copy(out_v, out_hbm.at[pl.ds(out0, R * F_OUT)])
            return 0

        lax.fori_loop(0, n_chunks, chunk_body, 0)

    return sc_call


def kernel(x, name_table, status_table, type_table, ability_table, item_table):
    B, T, F = x.shape
    n_rows = B * T
    sc_call = _make_sc_call(n_rows)
    out = sc_call(x.reshape(-1), name_table.reshape(-1),
                  status_table.reshape(-1), type_table.reshape(-1),
                  ability_table.reshape(-1), item_table.reshape(-1))
    return out.reshape(B, T, F_OUT)
